# baseline (speedup 1.0000x reference)
import jax
import jax.numpy as jnp
from jax import lax
from jax.experimental import pallas as pl
from jax.experimental.pallas import tpu as pltpu
from jax.experimental.pallas import tpu_sc as plsc

_NC = 2
_NS = 16
_NW = _NC * _NS
_L = 16

_B, _S, _H = 4, 8192, 1024
_SH = _S * _H
_PW = _SH // _NW
_CE = 32768
_NCHUNK = _PW // _CE


def _sc_body(x_hbm, pe_hbm, out_hbm, pe_v, x_v):
    cid = lax.axis_index("c")
    sid = lax.axis_index("s")
    wid = sid * _NC + cid
    base = wid * _PW
    for ci in range(_NCHUNK):
        off = base + ci * _CE
        pltpu.sync_copy(pe_hbm.at[pl.ds(off, _CE)], pe_v)
        for b in range(_B):
            xoff = b * _SH + off
            pltpu.sync_copy(x_hbm.at[pl.ds(xoff, _CE)], x_v)

            @plsc.parallel_loop(0, _CE // _L, unroll=8)
            def _(i):
                sl = pl.ds(i * _L, _L)
                plsc.addupdate(x_v.at[sl], pe_v[sl])

            pltpu.sync_copy(x_v, out_hbm.at[pl.ds(xoff, _CE)])


def kernel(x, pe):
    B, S, H = x.shape
    x_flat = x.reshape(B * S * H)
    pe_flat = pe.reshape(S * H)
    mesh = plsc.VectorSubcoreMesh(core_axis_name="c", subcore_axis_name="s")
    out = pl.kernel(
        _sc_body,
        out_type=jax.ShapeDtypeStruct((B * S * H,), jnp.float32),
        mesh=mesh,
        scratch_types=[
            pltpu.VMEM((_CE,), jnp.float32),
            pltpu.VMEM((_CE,), jnp.float32),
        ],
    )(x_flat, pe_flat)
    return out.reshape(B, S, H)


# speedup vs baseline: 1.1760x; 1.1760x over previous
import jax
import jax.numpy as jnp
from jax import lax
from jax.experimental import pallas as pl
from jax.experimental.pallas import tpu as pltpu
from jax.experimental.pallas import tpu_sc as plsc

_NC = 2
_NS = 16
_NW = _NC * _NS
_L = 16

_B, _S, _H = 4, 8192, 1024
_SH = _S * _H
_PW = _SH // _NW
_CE = 32768
_NCHUNK = _PW // _CE
_ITEMS = _NCHUNK * _B


def _sc_body(x_hbm, pe_hbm, out_hbm, pe_v, x0_v, x1_v,
             sem_in0, sem_in1, sem_out0, sem_out1):
    cid = lax.axis_index("c")
    sid = lax.axis_index("s")
    wid = sid * _NC + cid
    base = wid * _PW

    x_bufs = (x0_v, x1_v)
    sems_in = (sem_in0, sem_in1)
    sems_out = (sem_out0, sem_out1)

    def x_slice(k):
        ci, b = divmod(k, _B)
        off = base + ci * _CE
        return pl.ds(b * _SH + off, _CE)

    descs_in = [None, None]
    descs_out = [None, None]

    descs_in[0] = pltpu.async_copy(x_hbm.at[x_slice(0)], x_bufs[0], sems_in[0])

    for k in range(_ITEMS):
        ci, b = divmod(k, _B)
        buf = k % 2
        if b == 0:
            pltpu.sync_copy(pe_hbm.at[pl.ds(base + ci * _CE, _CE)], pe_v)
        if k + 1 < _ITEMS:
            nbuf = (k + 1) % 2
            if descs_out[nbuf] is not None:
                descs_out[nbuf].wait()
                descs_out[nbuf] = None
            descs_in[nbuf] = pltpu.async_copy(
                x_hbm.at[x_slice(k + 1)], x_bufs[nbuf], sems_in[nbuf])
        descs_in[buf].wait()
        descs_in[buf] = None

        x_v = x_bufs[buf]

        @plsc.parallel_loop(0, _CE // _L, unroll=8)
        def _(i):
            sl = pl.ds(i * _L, _L)
            plsc.addupdate(x_v.at[sl], pe_v[sl])

        descs_out[buf] = pltpu.async_copy(
            x_v, out_hbm.at[x_slice(k)], sems_out[buf])

    for buf in range(2):
        if descs_out[buf] is not None:
            descs_out[buf].wait()


def kernel(x, pe):
    B, S, H = x.shape
    x_flat = x.reshape(B * S * H)
    pe_flat = pe.reshape(S * H)
    mesh = plsc.VectorSubcoreMesh(core_axis_name="c", subcore_axis_name="s")
    out = pl.kernel(
        _sc_body,
        out_type=jax.ShapeDtypeStruct((B * S * H,), jnp.float32),
        mesh=mesh,
        scratch_types=[
            pltpu.VMEM((_CE,), jnp.float32),
            pltpu.VMEM((_CE,), jnp.float32),
            pltpu.VMEM((_CE,), jnp.float32),
            pltpu.SemaphoreType.DMA,
            pltpu.SemaphoreType.DMA,
            pltpu.SemaphoreType.DMA,
            pltpu.SemaphoreType.DMA,
        ],
    )(x_flat, pe_flat)
    return out.reshape(B, S, H)


# speedup vs baseline: 3.1912x; 2.7137x over previous
import jax
import jax.numpy as jnp
from jax import lax
from jax.experimental import pallas as pl
from jax.experimental.pallas import tpu as pltpu
from jax.experimental.pallas import tpu_sc as plsc

_NC = 2
_NS = 16
_NW = _NC * _NS
_L = 16

_B, _S, _H = 4, 8192, 1024
_RW = _S // _NW
_CR = 32
_NCHUNK = _RW // _CR
_ITEMS = _NCHUNK * _B


def _sc_body(x_hbm, pe_hbm, out_hbm, pe_v, x0_v, x1_v,
             sem_in0, sem_in1, sem_out0, sem_out1):
    cid = lax.axis_index("c")
    sid = lax.axis_index("s")
    wid = sid * _NC + cid
    base = wid * _RW

    x_bufs = (x0_v, x1_v)
    sems_in = (sem_in0, sem_in1)
    sems_out = (sem_out0, sem_out1)

    def rows(k):
        ci = k // _B
        return pl.ds(base + ci * _CR, _CR)

    def batch(k):
        return k % _B

    descs_in = [None, None]
    descs_out = [None, None]

    descs_in[0] = pltpu.async_copy(
        x_hbm.at[batch(0), rows(0)], x_bufs[0], sems_in[0])

    for k in range(_ITEMS):
        ci, b = divmod(k, _B)
        buf = k % 2
        if b == 0:
            pltpu.sync_copy(pe_hbm.at[rows(k)], pe_v)
        if k + 1 < _ITEMS:
            nbuf = (k + 1) % 2
            if descs_out[nbuf] is not None:
                descs_out[nbuf].wait()
                descs_out[nbuf] = None
            descs_in[nbuf] = pltpu.async_copy(
                x_hbm.at[batch(k + 1), rows(k + 1)], x_bufs[nbuf],
                sems_in[nbuf])
        descs_in[buf].wait()
        descs_in[buf] = None

        x_v = x_bufs[buf]

        @plsc.parallel_loop(0, _CR * _H // _L, unroll=8)
        def _(i):
            r = i // (_H // _L)
            c = lax.rem(i * _L, _H)
            plsc.addupdate(x_v.at[r, pl.ds(c, _L)], pe_v[r, pl.ds(c, _L)])

        descs_out[buf] = pltpu.async_copy(
            x_v, out_hbm.at[batch(k), rows(k)], sems_out[buf])

    for buf in range(2):
        if descs_out[buf] is not None:
            descs_out[buf].wait()


def kernel(x, pe):
    B, S, H = x.shape
    mesh = plsc.VectorSubcoreMesh(core_axis_name="c", subcore_axis_name="s")
    out = pl.kernel(
        _sc_body,
        out_type=jax.ShapeDtypeStruct((B, S, H), jnp.float32),
        mesh=mesh,
        scratch_types=[
            pltpu.VMEM((_CR, _H), jnp.float32),
            pltpu.VMEM((_CR, _H), jnp.float32),
            pltpu.VMEM((_CR, _H), jnp.float32),
            pltpu.SemaphoreType.DMA,
            pltpu.SemaphoreType.DMA,
            pltpu.SemaphoreType.DMA,
            pltpu.SemaphoreType.DMA,
        ],
    )(x, pe)
    return out
